# baseline (speedup 1.0000x reference)
import functools

import jax
import jax.numpy as jnp
from jax import lax
from jax.experimental import pallas as pl
from jax.experimental.pallas import tpu as pltpu
from jax.experimental.pallas import tpu_sc as plsc

N = 1048576
X_LIM, Y_LIM, Z_LIM = 511, 511, 255

NC, NS = 2, 16
NW = NC * NS
PW = N // NW
S = 8192
NSUB = PW // S
VPC = S // 16
C = 512

_mesh = plsc.VectorSubcoreMesh(core_axis_name="c", subcore_axis_name="s")


@functools.partial(
    pl.kernel,
    mesh=_mesh,
    compiler_params=pltpu.CompilerParams(needs_layout_passes=False),
    out_type=jax.ShapeDtypeStruct((4 * N,), jnp.float32),
    scratch_types=[
        pltpu.VMEM((S,), jnp.float32),
        pltpu.VMEM((S,), jnp.float32),
        pltpu.VMEM((S,), jnp.float32),
        pltpu.VMEM((S + 16,), jnp.int32),
        pltpu.VMEM((S + 16,), jnp.int32),
        pltpu.VMEM((S,), jnp.float32),
        pltpu.VMEM((4 * S,), jnp.float32),
        pltpu.SemaphoreType.DMA,
    ],
)
def _ct_gather(xyz_hbm, img_hbm, out_hbm, x_v, y_v, z_v, cidx_v, cpos_v,
               sig_v, out_v, sem):
    wid = lax.axis_index("s") * NC + lax.axis_index("c")
    iota = lax.iota(jnp.int32, 16)
    ones16 = jnp.full((16,), 1.0, jnp.float32)
    zeros16 = jnp.full((16,), 0.0, jnp.float32)
    zeros16i = jnp.full((16,), 0, jnp.int32)
    base = wid * PW

    def _fill(g, c):
        out_v[pl.ds(g * 16, 16)] = ones16
        return c
    lax.fori_loop(0, (4 * S) // 16, _fill, 0)

    def _fill0(g, c):
        cidx_v[pl.ds(g * 16, 16)] = zeros16i
        return c
    lax.fori_loop(0, (S + 16) // 16, _fill0, 0)

    def _sub(sub, c):
        sbase = base + ((sub + wid) % NSUB) * S
        pltpu.sync_copy(xyz_hbm.at[pl.ds(sbase, S)], x_v)
        pltpu.sync_copy(xyz_hbm.at[pl.ds(N + sbase, S)], y_v)
        pltpu.sync_copy(xyz_hbm.at[pl.ds(2 * N + sbase, S)], z_v)

        @plsc.parallel_loop(0, VPC, unroll=4,
                            carry=jnp.zeros((16,), jnp.int32))
        def _comp(g, off_vec):
            x = x_v[pl.ds(g * 16, 16)]
            y = y_v[pl.ds(g * 16, 16)]
            z = z_v[pl.ds(g * 16, 16)]
            ix = ((x + 1.0) * 255.5).astype(jnp.int32)
            iy = ((y + 1.0) * 255.5).astype(jnp.int32)
            iz = ((z + 1.0) * 127.5).astype(jnp.int32)
            good = ((ix.astype(jnp.uint32) <= X_LIM)
                    & (iy.astype(jnp.uint32) <= Y_LIM)
                    & (iz.astype(jnp.uint32) <= Z_LIM))
            phys = ((ix << 17) + ((iy >> 3) << 11) + ((iz >> 7) << 10)
                    + ((iy & 7) << 7) + (iz & 127))
            goodi = good.astype(jnp.int32)
            rank = jnp.cumsum(goodi) - goodi
            addr = off_vec + rank
            plsc.store_scatter(cidx_v, [addr], phys, mask=good)
            plsc.store_scatter(cpos_v, [addr], g * 16 + iota, mask=good)
            return off_vec + plsc.all_reduce_population_count(good)
        n_valid = jnp.max(_comp)

        @plsc.parallel_loop(0, VPC, unroll=4)
        def _zero(g):
            b = g * 16
            out_v[pl.ds((b >> 7) * 512 + 384 + (b & 127), 16)] = zeros16

        nch = (n_valid + (C - 1)) // C

        def _fire(j, cc):
            pltpu.async_copy(img_hbm.at[cidx_v.at[pl.ds(j * C, C)]],
                             sig_v.at[pl.ds(j * C, C)], sem)
            return cc
        lax.fori_loop(0, nch, _fire, 0)

        def _drain(j, cc):
            pltpu.make_async_copy(img_hbm.at[cidx_v.at[pl.ds(j * C, C)]],
                                  sig_v.at[pl.ds(j * C, C)], sem).wait()
            return cc
        lax.fori_loop(0, nch, _drain, 0)

        nvec = (n_valid + 15) >> 4

        @plsc.parallel_loop(0, nvec, unroll=2)
        def _outp(g):
            sv = sig_v[pl.ds(g * 16, 16)]
            pos = cpos_v[pl.ds(g * 16, 16)]
            slot = ((pos >> 7) << 9) + 384 + (pos & 127)
            ok = (g * 16 + iota) < n_valid
            plsc.store_scatter(out_v, [slot], sv, mask=ok)

        pltpu.sync_copy(out_v, out_hbm.at[pl.ds(4 * sbase, 4 * S)])
        return c
    lax.fori_loop(0, NSUB, _sub, 0)


def kernel(xyz, img):
    xyz_planar = jnp.transpose(xyz, (2, 0, 1)).reshape(3 * N)
    img_tiled = (img.reshape(512, 64, 8, 2, 128)
                 .transpose(0, 1, 3, 2, 4).reshape(64 * N))
    out = _ct_gather(xyz_planar, img_tiled)
    return out.reshape(N // 128, 4, 128).transpose(0, 2, 1).reshape(1, N, 4)


# speedup vs baseline: 1.0233x; 1.0233x over previous
import functools

import jax
import jax.numpy as jnp
from jax import lax
from jax.experimental import pallas as pl
from jax.experimental.pallas import tpu as pltpu
from jax.experimental.pallas import tpu_sc as plsc

N = 1048576
X_LIM, Y_LIM, Z_LIM = 511, 511, 255

NC, NS = 2, 16
NW = NC * NS
PW = N // NW
S = 8192
NSUB = PW // S
VPC = S // 16
C = 512

_mesh = plsc.VectorSubcoreMesh(core_axis_name="c", subcore_axis_name="s")


@functools.partial(
    pl.kernel,
    mesh=_mesh,
    compiler_params=pltpu.CompilerParams(needs_layout_passes=False),
    out_type=jax.ShapeDtypeStruct((4 * N,), jnp.float32),
    scratch_types=[
        pltpu.VMEM((S,), jnp.float32),
        pltpu.VMEM((S,), jnp.float32),
        pltpu.VMEM((S,), jnp.float32),
        pltpu.VMEM((S + 16,), jnp.int32),
        pltpu.VMEM((S + 16,), jnp.int32),
        pltpu.VMEM((S,), jnp.float32),
        pltpu.VMEM((4 * S,), jnp.float32),
        pltpu.SemaphoreType.DMA,
        pltpu.SemaphoreType.DMA,
        pltpu.SemaphoreType.DMA,
    ],
)
def _ct_gather(xyz_hbm, img_hbm, out_hbm, x_v, y_v, z_v, cidx_v, cpos_v,
               sig_v, out_v, sem, sem_in, sem_out):
    wid = lax.axis_index("s") * NC + lax.axis_index("c")
    iota = lax.iota(jnp.int32, 16)
    ones16 = jnp.full((16,), 1.0, jnp.float32)
    zeros16 = jnp.full((16,), 0.0, jnp.float32)
    zeros16i = jnp.full((16,), 0, jnp.int32)
    base = wid * PW

    def _fill(g, c):
        out_v[pl.ds(g * 16, 16)] = ones16
        return c
    lax.fori_loop(0, (4 * S) // 16, _fill, 0)

    def _fill0(g, c):
        cidx_v[pl.ds(g * 16, 16)] = zeros16i
        return c
    lax.fori_loop(0, (S + 16) // 16, _fill0, 0)

    def _sub(sub, c):
        sbase = base + sub * S
        pltpu.async_copy(xyz_hbm.at[pl.ds(sbase, S)], x_v, sem_in)
        pltpu.async_copy(xyz_hbm.at[pl.ds(N + sbase, S)], y_v, sem_in)
        pltpu.async_copy(xyz_hbm.at[pl.ds(2 * N + sbase, S)], z_v, sem_in)
        pltpu.make_async_copy(xyz_hbm.at[pl.ds(sbase, S)], x_v, sem_in).wait()
        pltpu.make_async_copy(xyz_hbm.at[pl.ds(N + sbase, S)], y_v,
                              sem_in).wait()
        pltpu.make_async_copy(xyz_hbm.at[pl.ds(2 * N + sbase, S)], z_v,
                              sem_in).wait()

        @plsc.parallel_loop(0, VPC, unroll=4,
                            carry=jnp.zeros((16,), jnp.int32))
        def _comp(g, off_vec):
            x = x_v[pl.ds(g * 16, 16)]
            y = y_v[pl.ds(g * 16, 16)]
            z = z_v[pl.ds(g * 16, 16)]
            ix = ((x + 1.0) * 255.5).astype(jnp.int32)
            iy = ((y + 1.0) * 255.5).astype(jnp.int32)
            iz = ((z + 1.0) * 127.5).astype(jnp.int32)
            good = ((ix.astype(jnp.uint32) <= X_LIM)
                    & (iy.astype(jnp.uint32) <= Y_LIM)
                    & (iz.astype(jnp.uint32) <= Z_LIM))
            phys = ((ix << 17) + ((iy >> 3) << 11) + ((iz >> 7) << 10)
                    + ((iy & 7) << 7) + (iz & 127))
            goodi = good.astype(jnp.int32)
            rank = jnp.cumsum(goodi) - goodi
            addr = off_vec + rank
            plsc.store_scatter(cidx_v, [addr], phys, mask=good)
            plsc.store_scatter(cpos_v, [addr], g * 16 + iota, mask=good)
            return off_vec + plsc.all_reduce_population_count(good)
        n_valid = jnp.max(_comp)

        @pl.when(sub > 0)
        def _():
            pltpu.make_async_copy(
                out_v, out_hbm.at[pl.ds(4 * (sbase - S), 4 * S)],
                sem_out).wait()

        @plsc.parallel_loop(0, VPC, unroll=4)
        def _zero(g):
            b = g * 16
            out_v[pl.ds((b >> 7) * 512 + 384 + (b & 127), 16)] = zeros16

        nch = (n_valid + (C - 1)) // C

        def _fire(j, cc):
            pltpu.async_copy(img_hbm.at[cidx_v.at[pl.ds(j * C, C)]],
                             sig_v.at[pl.ds(j * C, C)], sem)
            return cc
        lax.fori_loop(0, nch, _fire, 0)

        def _drain(j, cc):
            pltpu.make_async_copy(img_hbm.at[cidx_v.at[pl.ds(j * C, C)]],
                                  sig_v.at[pl.ds(j * C, C)], sem).wait()
            return cc
        lax.fori_loop(0, nch, _drain, 0)

        nvec = (n_valid + 15) >> 4

        @plsc.parallel_loop(0, nvec, unroll=2)
        def _outp(g):
            sv = sig_v[pl.ds(g * 16, 16)]
            pos = cpos_v[pl.ds(g * 16, 16)]
            slot = ((pos >> 7) << 9) + 384 + (pos & 127)
            ok = (g * 16 + iota) < n_valid
            plsc.store_scatter(out_v, [slot], sv, mask=ok)

        pltpu.async_copy(out_v, out_hbm.at[pl.ds(4 * sbase, 4 * S)], sem_out)
        return c
    lax.fori_loop(0, NSUB, _sub, 0)
    pltpu.make_async_copy(
        out_v, out_hbm.at[pl.ds(4 * (base + (NSUB - 1) * S), 4 * S)],
        sem_out).wait()


def kernel(xyz, img):
    xyz_planar = jnp.transpose(xyz, (2, 0, 1)).reshape(3 * N)
    img_tiled = (img.reshape(512, 64, 8, 2, 128)
                 .transpose(0, 1, 3, 2, 4).reshape(64 * N))
    out = _ct_gather(xyz_planar, img_tiled)
    return out.reshape(N // 128, 4, 128).transpose(0, 2, 1).reshape(1, N, 4)
